# baseline (speedup 1.0000x reference)
import functools

import jax
import jax.numpy as jnp
from jax import lax
from jax.experimental import pallas as pl
from jax.experimental.pallas import tpu as pltpu
from jax.experimental.pallas import tpu_sc as plsc


def _sc_embed(ids_flat, token_table, position_table, *, B, L, D):
    NC, NS = 2, 16
    NW = NC * NS
    n_rows = (B * L) // NW
    CH = 128
    NCH = n_rows // CH
    NB = 3

    mesh = plsc.VectorSubcoreMesh(core_axis_name="c", subcore_axis_name="s")

    @functools.partial(
        pl.kernel,
        mesh=mesh,
        out_type=jax.ShapeDtypeStruct((B * L, D), jnp.float32),
        scratch_types=[
            pltpu.VMEM((n_rows,), jnp.int32),
            pltpu.VMEM((L, D), jnp.float32),
        ] + [pltpu.VMEM((CH, D), jnp.float32) for _ in range(2 * NB)]
          + [pltpu.SemaphoreType.DMA for _ in range(2 * NB)],
    )
    def k(ids_hbm, tbl_hbm, pos_hbm, out_hbm, idx_v, pos_v, *rest):
        gbuf = rest[:NB]
        obuf = rest[NB:2 * NB]
        gsem = rest[2 * NB:3 * NB]
        ssem = rest[3 * NB:]

        wid = lax.axis_index("s") * NC + lax.axis_index("c")
        base = wid * n_rows
        pltpu.sync_copy(ids_hbm.at[pl.ds(base, n_rows)], idx_v)
        pltpu.sync_copy(pos_hbm.at[pl.ds(0, L)], pos_v)

        def issue_gather(c):
            b = c % NB
            return pltpu.async_copy(
                tbl_hbm.at[idx_v.at[pl.ds(c * CH, CH)]], gbuf[b], gsem[b])

        gathers = {c: issue_gather(c) for c in range(NB)}
        scatters = {}

        for c in range(NCH):
            b = c % NB
            gathers.pop(c).wait()
            if c >= NB:
                scatters.pop(c - NB).wait()

            g, o, off = gbuf[b], obuf[b], c * CH

            def add_row(i, carry, g=g, o=o, off=off):
                l = lax.rem(off + i, L)
                for j in range(D // 16):
                    sl = pl.ds(j * 16, 16)
                    o[i, sl] = g[i, sl] + pos_v[l, sl]
                return carry

            lax.fori_loop(0, CH, add_row, 0)

            scatters[c] = pltpu.async_copy(
                o, out_hbm.at[pl.ds(base + off, CH)], ssem[b])
            if c + NB < NCH:
                gathers[c + NB] = issue_gather(c + NB)

        for c in sorted(scatters):
            scatters[c].wait()

    return k(ids_flat, token_table, position_table)


def kernel(input_ids, token_table, position_table):
    B, L = input_ids.shape
    _, D = token_table.shape
    ids_flat = input_ids.reshape(B * L).astype(jnp.int32)
    out = _sc_embed(ids_flat, token_table.astype(jnp.float32),
                    position_table.astype(jnp.float32), B=B, L=L, D=D)
    return out.reshape(B, L, D)


# speedup vs baseline: 2.8004x; 2.8004x over previous
import functools

import jax
import jax.numpy as jnp
from jax import lax
from jax.experimental import pallas as pl
from jax.experimental.pallas import tpu as pltpu
from jax.experimental.pallas import tpu_sc as plsc


def _sc_embed(ids_flat, token_table, position_table, *, B, L, D):
    NC, NS = 2, 16
    NW = NC * NS
    n_rows = (B * L) // NW
    CH = 128
    NCH = n_rows // CH
    NB = 3

    mesh = plsc.VectorSubcoreMesh(core_axis_name="c", subcore_axis_name="s")

    @functools.partial(
        pl.kernel,
        mesh=mesh,
        out_type=jax.ShapeDtypeStruct((B * L, D), jnp.float32),
        scratch_types=[
            pltpu.VMEM((n_rows,), jnp.int32),
            pltpu.VMEM((L, D), jnp.float32),
        ] + [pltpu.VMEM((CH, D), jnp.float32) for _ in range(2 * NB)]
          + [pltpu.SemaphoreType.DMA for _ in range(2 * NB + 2)],
    )
    def k(ids_hbm, tbl_hbm, pos_hbm, out_hbm, idx_v, pos_v, *rest):
        gbuf = rest[:NB]
        obuf = rest[NB:2 * NB]
        gsem = rest[2 * NB:3 * NB]
        ssem = rest[3 * NB:4 * NB]
        isem, psem = rest[4 * NB], rest[4 * NB + 1]

        wid = lax.axis_index("s") * NC + lax.axis_index("c")
        base = wid * n_rows
        idx_cp = pltpu.async_copy(ids_hbm.at[pl.ds(base, n_rows)], idx_v, isem)
        pos_cp = pltpu.async_copy(pos_hbm.at[pl.ds(0, L)], pos_v, psem)
        idx_cp.wait()

        def issue_gather(c):
            b = c % NB
            return pltpu.async_copy(
                tbl_hbm.at[idx_v.at[pl.ds(c * CH, CH)]], gbuf[b], gsem[b])

        gathers = {c: issue_gather(c) for c in range(NB)}
        scatters = {}
        pos_cp.wait()

        for c in range(NCH):
            b = c % NB
            gathers.pop(c).wait()
            if c >= NB:
                scatters.pop(c - NB).wait()

            g, o, off = gbuf[b], obuf[b], c * CH
            l0 = off % L
            n1 = min(CH, L - l0)

            def add_run(i0, cnt, lbase, g=g, o=o):
                def add_row(i, carry):
                    for j in range(D // 16):
                        sl = pl.ds(j * 16, 16)
                        o[i0 + i, sl] = g[i0 + i, sl] + pos_v[lbase + i, sl]
                    return carry
                lax.fori_loop(0, cnt, add_row, 0)

            add_run(0, n1, l0)
            if n1 < CH:
                add_run(n1, CH - n1, 0)

            scatters[c] = pltpu.async_copy(
                o, out_hbm.at[pl.ds(base + off, CH)], ssem[b])
            if c + NB < NCH:
                gathers[c + NB] = issue_gather(c + NB)

        for c in sorted(scatters):
            scatters[c].wait()

    return k(ids_flat, token_table, position_table)


def kernel(input_ids, token_table, position_table):
    B, L = input_ids.shape
    _, D = token_table.shape
    ids_flat = input_ids.reshape(B * L).astype(jnp.int32)
    out = _sc_embed(ids_flat, token_table.astype(jnp.float32),
                    position_table.astype(jnp.float32), B=B, L=L, D=D)
    return out.reshape(B, L, D)


# speedup vs baseline: 2.8976x; 1.0347x over previous
import functools

import jax
import jax.numpy as jnp
from jax import lax
from jax.experimental import pallas as pl
from jax.experimental.pallas import tpu as pltpu
from jax.experimental.pallas import tpu_sc as plsc


def _sc_embed(ids_flat, token_table, position_table, *, B, L, D):
    NC, NS = 2, 16
    NW = NC * NS
    BPW = B // NW
    n_rows = BPW * L
    NBUF = 3

    mesh = plsc.VectorSubcoreMesh(core_axis_name="c", subcore_axis_name="s")

    @functools.partial(
        pl.kernel,
        mesh=mesh,
        out_type=jax.ShapeDtypeStruct((B * L, D), jnp.float32),
        scratch_types=[
            pltpu.VMEM((n_rows,), jnp.int32),
            pltpu.VMEM((L, D), jnp.float32),
        ] + [pltpu.VMEM((L, D), jnp.float32) for _ in range(NBUF)]
          + [pltpu.SemaphoreType.DMA for _ in range(2 * NBUF + 2)],
    )
    def k(ids_hbm, tbl_hbm, pos_hbm, out_hbm, idx_v, pos_v, *rest):
        bufs = rest[:NBUF]
        gsem = rest[NBUF:2 * NBUF]
        ssem = rest[2 * NBUF:3 * NBUF]
        isem, psem = rest[3 * NBUF], rest[3 * NBUF + 1]

        wid = lax.axis_index("s") * NC + lax.axis_index("c")
        base = wid * n_rows
        idx_cp = pltpu.async_copy(ids_hbm.at[pl.ds(base, n_rows)], idx_v, isem)
        pos_cp = pltpu.async_copy(pos_hbm.at[pl.ds(0, L)], pos_v, psem)
        idx_cp.wait()

        def issue_gather(c):
            b = c % NBUF
            off = c * L
            cp1 = pltpu.async_copy(
                tbl_hbm.at[idx_v.at[pl.ds(off, 128)]],
                bufs[b].at[pl.ds(0, 128)], gsem[b])
            cp2 = pltpu.async_copy(
                tbl_hbm.at[idx_v.at[pl.ds(off + 128, L - 128)]],
                bufs[b].at[pl.ds(128, L - 128)], gsem[b])
            return (cp1, cp2)

        gathers = {0: issue_gather(0), 1: issue_gather(1)}
        scatters = {}
        pos_cp.wait()

        for c in range(BPW):
            b = c % NBUF
            cp1, cp2 = gathers.pop(c)
            cp1.wait()
            cp2.wait()

            buf = bufs[b]

            def add_row(l, carry, buf=buf):
                for j in range(D // 16):
                    sl = pl.ds(j * 16, 16)
                    buf[l, sl] = buf[l, sl] + pos_v[l, sl]
                return carry

            lax.fori_loop(0, L, add_row, 0)

            scatters[c] = pltpu.async_copy(
                buf, out_hbm.at[pl.ds(base + c * L, L)], ssem[b])

            if c + 2 < BPW:
                if c >= 1:
                    scatters.pop(c - 1).wait()
                gathers[c + 2] = issue_gather(c + 2)

        for c in sorted(scatters):
            scatters[c].wait()

    return k(ids_flat, token_table, position_table)


def kernel(input_ids, token_table, position_table):
    B, L = input_ids.shape
    _, D = token_table.shape
    ids_flat = input_ids.reshape(B * L).astype(jnp.int32)
    out = _sc_embed(ids_flat, token_table.astype(jnp.float32),
                    position_table.astype(jnp.float32), B=B, L=L, D=D)
    return out.reshape(B, L, D)
